# Initial kernel scaffold; baseline (speedup 1.0000x reference)
import jax
import jax.numpy as jnp
from jax.experimental import pallas as pl


def kernel(x, edge_index, batch, shape_emb, color_emb, W1l, b1, W1r, W2l, b2, W2r, Wc, bc):
    raise NotImplementedError("write your pallas kernel here")


# speedup vs baseline: 10.0880x; 10.0880x over previous
import dataclasses
import functools

import jax
import jax.numpy as jnp
from jax import lax
from jax.experimental import pallas as pl
from jax.experimental.pallas import tpu as pltpu
from jax.experimental.pallas import tpu_sc as plsc

N = 100000
E = 1600000
G = 1024
F = 32
NCLS = 32

NSC = 2
NSUB = 16
NW = NSC * NSUB

BN = 2048
NBLK = 49
N_pad = BN * NBLK
CH = N_pad // NW
SUB = CH // 2

H = N_pad // NSC
NDUMP = 128
H2 = H + NDUMP
ZCH = H2 // NSUB

K = 512
EC = 100352
E_pad = EC * NSUB

_mesh = plsc.VectorSubcoreMesh(core_axis_name="core", subcore_axis_name="subcore")

_sc_params = pltpu.CompilerParams(
    needs_layout_passes=False, use_tc_tiling_on_sc=False)


def _embed_body(x0_hbm, x1_hbm, se_hbm, ce_hbm, h0_hbm, x0_v, x1_v, se_v, ce_v, hb_v):
    wid = lax.axis_index("subcore") * NSC + lax.axis_index("core")
    base = wid * CH
    pltpu.sync_copy(x0_hbm.at[pl.ds(base, CH)], x0_v)
    pltpu.sync_copy(x1_hbm.at[pl.ds(base, CH)], x1_v)
    pltpu.sync_copy(se_hbm, se_v)
    pltpu.sync_copy(ce_hbm, ce_v)
    iota = lax.iota(jnp.int32, 16)
    for half in range(2):
        @pl.loop(0, SUB, step=16)
        def _(v):
            row0 = half * SUB + v
            xv0 = x0_v[pl.ds(row0, 16)]
            xv1 = x1_v[pl.ds(row0, 16)]
            rows = v + iota
            for j in range(16):
                cj = jnp.full((16,), j, jnp.int32)
                s_col = plsc.load_gather(se_v, [xv0, cj])
                plsc.store_scatter(hb_v, [rows, cj], s_col)
                c_col = plsc.load_gather(ce_v, [xv1, cj])
                plsc.store_scatter(hb_v, [rows, cj + 16], c_col)
        pltpu.sync_copy(hb_v, h0_hbm.at[pl.ds(base + half * SUB, SUB)])


@jax.jit
def _embed(x0, x1, se, ce):
    kfn = pl.kernel(
        _embed_body,
        out_type=jax.ShapeDtypeStruct((N_pad, F), jnp.float32),
        mesh=_mesh,
        compiler_params=_sc_params,
        scratch_types=[
            pltpu.VMEM((CH,), jnp.int32),
            pltpu.VMEM((CH,), jnp.int32),
            pltpu.VMEM((128, 16), jnp.float32),
            pltpu.VMEM((128, 16), jnp.float32),
            pltpu.VMEM((SUB, F), jnp.float32),
        ],
    )
    return kfn(x0, x1, se, ce)


def _agg_body(with_deg, *args):
    if with_deg:
        (h_hbm, s_hbm, d_hbm, z2_hbm, z1_hbm, agg_hbm, deg_hbm,
         sv, dv, iv, rows_v, ones_v, acc, accd) = args
    else:
        (h_hbm, s_hbm, d_hbm, z2_hbm, agg_hbm,
         sv, dv, iv, rows_v, acc) = args
    core = lax.axis_index("core")
    sub = lax.axis_index("subcore")
    pltpu.sync_copy(z2_hbm, acc.at[pl.ds(sub * ZCH, ZCH)])
    if with_deg:
        pltpu.sync_copy(z1_hbm, accd.at[pl.ds(sub * ZCH, ZCH)])

        @pl.loop(0, K, step=16)
        def _(q):
            ones_v[pl.ds(q, 16)] = jnp.full((16,), 1.0, jnp.float32)

    plsc.subcore_barrier()
    half_base = core * H
    tile_edge_base = sub * EC

    @pl.loop(0, EC, step=K)
    def _(w):
        eb = tile_edge_base + w
        pltpu.sync_copy(s_hbm.at[pl.ds(eb, K)], sv)
        pltpu.sync_copy(d_hbm.at[pl.ds(eb, K)], dv)

        @pl.loop(0, K, step=16)
        def _(q):
            dl = dv[pl.ds(q, 16)] - half_base
            valid = (dl >= 0) & (dl < H)
            iv[pl.ds(q, 16)] = jnp.where(valid, dl, H + (dl & (NDUMP - 1)))

        pltpu.sync_copy(h_hbm.at[sv], rows_v)
        pltpu.sync_copy(rows_v, acc.at[iv], add=True)
        if with_deg:
            pltpu.sync_copy(ones_v, accd.at[iv], add=True)

    plsc.subcore_barrier()
    out_base = core * H + sub * (H // NSUB)
    pltpu.sync_copy(acc.at[pl.ds(sub * (H // NSUB), H // NSUB)],
                    agg_hbm.at[pl.ds(out_base, H // NSUB)])
    if with_deg:
        pltpu.sync_copy(accd.at[pl.ds(sub * (H // NSUB), H // NSUB)],
                        deg_hbm.at[pl.ds(out_base, H // NSUB)])


@jax.jit
def _aggregate_deg(h, srcp, dstp, z2, z1):
    kfn = pl.kernel(
        functools.partial(_agg_body, True),
        out_type=(jax.ShapeDtypeStruct((N_pad, F), jnp.float32),
                  jax.ShapeDtypeStruct((N_pad,), jnp.float32)),
        mesh=_mesh,
        compiler_params=_sc_params,
        scratch_types=[
            pltpu.VMEM((K,), jnp.int32),
            pltpu.VMEM((K,), jnp.int32),
            pltpu.VMEM((K,), jnp.int32),
            pltpu.VMEM((K, F), jnp.float32),
            pltpu.VMEM((K,), jnp.float32),
            pltpu.VMEM_SHARED((H2, F), jnp.float32),
            pltpu.VMEM_SHARED((H2,), jnp.float32),
        ],
    )
    return kfn(h, srcp, dstp, z2, z1)


@jax.jit
def _aggregate(h, srcp, dstp, z2):
    kfn = pl.kernel(
        functools.partial(_agg_body, False),
        out_type=jax.ShapeDtypeStruct((N_pad, F), jnp.float32),
        mesh=_mesh,
        compiler_params=_sc_params,
        scratch_types=[
            pltpu.VMEM((K,), jnp.int32),
            pltpu.VMEM((K,), jnp.int32),
            pltpu.VMEM((K,), jnp.int32),
            pltpu.VMEM((K, F), jnp.float32),
            pltpu.VMEM_SHARED((H2, F), jnp.float32),
        ],
    )
    return kfn(h, srcp, dstp, z2)


def _dense_body(agg_ref, deg_ref, h_ref, wl_ref, b_ref, wr_ref, out_ref):
    mean = agg_ref[...] / jnp.maximum(deg_ref[...], 1.0)[:, None]
    out = (lax.dot_general(mean, wl_ref[...], (((1,), (1,)), ((), ())),
                           preferred_element_type=jnp.float32)
           + lax.dot_general(h_ref[...], wr_ref[...], (((1,), (1,)), ((), ())),
                             preferred_element_type=jnp.float32)
           + b_ref[...])
    out_ref[...] = jnp.maximum(out, 0.0)


@jax.jit
def _dense(agg, deg, h, wl, b, wr):
    return pl.pallas_call(
        _dense_body,
        grid=(NBLK,),
        in_specs=[
            pl.BlockSpec((BN, F), lambda i: (i, 0)),
            pl.BlockSpec((BN,), lambda i: (i,)),
            pl.BlockSpec((BN, F), lambda i: (i, 0)),
            pl.BlockSpec((F, F), lambda i: (0, 0)),
            pl.BlockSpec((1, F), lambda i: (0, 0)),
            pl.BlockSpec((F, F), lambda i: (0, 0)),
        ],
        out_specs=pl.BlockSpec((BN, F), lambda i: (i, 0)),
        out_shape=jax.ShapeDtypeStruct((N_pad, F), jnp.float32),
    )(agg, deg, h, wl, b, wr)


def _pool_body(h_ref, batch_ref, wc_ref, bc_ref, out_ref, acc_s, acc_c):
    i = pl.program_id(0)

    @pl.when(i == 0)
    def _():
        acc_s[...] = jnp.zeros_like(acc_s)
        acc_c[...] = jnp.zeros_like(acc_c)

    ids = batch_ref[...]
    gi = lax.broadcasted_iota(jnp.int32, (G, BN), 0)
    oh = (gi == ids[None, :]).astype(jnp.float32)
    acc_s[...] += lax.dot_general(oh, h_ref[...], (((1,), (0,)), ((), ())),
                                  preferred_element_type=jnp.float32)
    acc_c[...] += jnp.sum(oh, axis=1)

    @pl.when(i == NBLK - 1)
    def _():
        hg = acc_s[...] / jnp.maximum(acc_c[...], 1.0)[:, None]
        out_ref[...] = lax.dot_general(hg, wc_ref[...], (((1,), (1,)), ((), ())),
                                       preferred_element_type=jnp.float32) + bc_ref[...]


@jax.jit
def _pool(h, batch, wc, bc):
    return pl.pallas_call(
        _pool_body,
        grid=(NBLK,),
        in_specs=[
            pl.BlockSpec((BN, F), lambda i: (i, 0)),
            pl.BlockSpec((BN,), lambda i: (i,)),
            pl.BlockSpec((NCLS, F), lambda i: (0, 0)),
            pl.BlockSpec((1, NCLS), lambda i: (0, 0)),
        ],
        out_specs=pl.BlockSpec((G, NCLS), lambda i: (0, 0)),
        out_shape=jax.ShapeDtypeStruct((G, NCLS), jnp.float32),
        scratch_shapes=[
            pltpu.VMEM((G, F), jnp.float32),
            pltpu.VMEM((G,), jnp.float32),
        ],
    )(h, batch, wc, bc)


def kernel(x, edge_index, batch, shape_emb, color_emb, W1l, b1, W1r, W2l, b2, W2r, Wc, bc):
    x = x.astype(jnp.int32)
    x0 = jnp.pad(x[:, 0], (0, N_pad - N))
    x1 = jnp.pad(x[:, 1], (0, N_pad - N))
    src = jnp.pad(edge_index[0], (0, E_pad - E))
    dst = jnp.pad(edge_index[1], (0, E_pad - E), constant_values=-1)
    batch_p = jnp.pad(batch, (0, N_pad - N), constant_values=G)
    z2 = jnp.zeros((ZCH, F), jnp.float32)
    z1 = jnp.zeros((ZCH,), jnp.float32)

    h0 = _embed(x0, x1, shape_emb, color_emb)
    a1, deg = _aggregate_deg(h0, src, dst, z2, z1)
    h1 = _dense(a1, deg, h0, W1l, b1.reshape(1, F), W1r)
    a2 = _aggregate(h1, src, dst, z2)
    h2 = _dense(a2, deg, h1, W2l, b2.reshape(1, F), W2r)
    return _pool(h2, batch_p, Wc, bc.reshape(1, NCLS))


# speedup vs baseline: 12.8057x; 1.2694x over previous
import dataclasses
import functools

import jax
import jax.numpy as jnp
from jax import lax
from jax.experimental import pallas as pl
from jax.experimental.pallas import tpu as pltpu
from jax.experimental.pallas import tpu_sc as plsc

N = 100000
E = 1600000
G = 1024
F = 32
NCLS = 32

NSC = 2
NSUB = 16
NW = NSC * NSUB

BN = 2048
NBLK = 49
N_pad = BN * NBLK
CH = N_pad // NW
SUB = CH // 2

H = N_pad // NSC
NDUMP = 128
H2 = H + NDUMP
ZCH = H2 // NSUB

K = 256
EC = 100352
E_pad = EC * NSUB

_mesh = plsc.VectorSubcoreMesh(core_axis_name="core", subcore_axis_name="subcore")

_sc_params = pltpu.CompilerParams(
    needs_layout_passes=False, use_tc_tiling_on_sc=False)


def _embed_body(x0_hbm, x1_hbm, se_hbm, ce_hbm, h0_hbm, x0_v, x1_v, se_v, ce_v, hb_v):
    wid = lax.axis_index("subcore") * NSC + lax.axis_index("core")
    base = wid * CH
    pltpu.sync_copy(x0_hbm.at[pl.ds(base, CH)], x0_v)
    pltpu.sync_copy(x1_hbm.at[pl.ds(base, CH)], x1_v)
    pltpu.sync_copy(se_hbm, se_v)
    pltpu.sync_copy(ce_hbm, ce_v)
    iota = lax.iota(jnp.int32, 16)
    for half in range(2):
        @pl.loop(0, SUB, step=16)
        def _(v):
            row0 = half * SUB + v
            xv0 = x0_v[pl.ds(row0, 16)]
            xv1 = x1_v[pl.ds(row0, 16)]
            rows = v + iota
            for j in range(16):
                cj = jnp.full((16,), j, jnp.int32)
                s_col = plsc.load_gather(se_v, [xv0, cj])
                plsc.store_scatter(hb_v, [rows, cj], s_col)
                c_col = plsc.load_gather(ce_v, [xv1, cj])
                plsc.store_scatter(hb_v, [rows, cj + 16], c_col)
        pltpu.sync_copy(hb_v, h0_hbm.at[pl.ds(base + half * SUB, SUB)])


@jax.jit
def _embed(x0, x1, se, ce):
    kfn = pl.kernel(
        _embed_body,
        out_type=jax.ShapeDtypeStruct((N_pad, F), jnp.float32),
        mesh=_mesh,
        compiler_params=_sc_params,
        scratch_types=[
            pltpu.VMEM((CH,), jnp.int32),
            pltpu.VMEM((CH,), jnp.int32),
            pltpu.VMEM((128, 16), jnp.float32),
            pltpu.VMEM((128, 16), jnp.float32),
            pltpu.VMEM((SUB, F), jnp.float32),
        ],
    )
    return kfn(x0, x1, se, ce)


def _agg_body(with_deg, *args):
    if with_deg:
        (h_hbm, s_hbm, d_hbm, z2_hbm, z1_hbm, agg_hbm, deg_hbm,
         sv0, dv0, iv0, rows0, sv1, dv1, iv1, rows1,
         lsem0, lsem1, gsem, ssem0, ssem1, ones_v, acc, accd) = args
    else:
        (h_hbm, s_hbm, d_hbm, z2_hbm, agg_hbm,
         sv0, dv0, iv0, rows0, sv1, dv1, iv1, rows1,
         lsem0, lsem1, gsem, ssem0, ssem1, acc) = args
    svs, dvs, ivs, rows_ = (sv0, sv1), (dv0, dv1), (iv0, iv1), (rows0, rows1)
    lsems, ssems = (lsem0, lsem1), (ssem0, ssem1)
    core = lax.axis_index("core")
    sub = lax.axis_index("subcore")
    pltpu.sync_copy(z2_hbm, acc.at[pl.ds(sub * ZCH, ZCH)])
    if with_deg:
        pltpu.sync_copy(z1_hbm, accd.at[pl.ds(sub * ZCH, ZCH)])

        @pl.loop(0, K, step=16)
        def _(q):
            ones_v[pl.ds(q, 16)] = jnp.full((16,), 1.0, jnp.float32)

    plsc.subcore_barrier()
    half_base = core * H
    tile_edge_base = sub * EC
    nw = EC // K

    def load(w, p):
        eb = tile_edge_base + w * K
        pltpu.async_copy(s_hbm.at[pl.ds(eb, K)], svs[p], lsems[p])
        pltpu.async_copy(d_hbm.at[pl.ds(eb, K)], dvs[p], lsems[p])

    def wait_load(p):
        pltpu.make_async_copy(s_hbm.at[pl.ds(0, K)], svs[p], lsems[p]).wait()
        pltpu.make_async_copy(d_hbm.at[pl.ds(0, K)], dvs[p], lsems[p]).wait()

    def wait_scatter(p):
        pltpu.make_async_copy(rows_[p], acc.at[ivs[p]], ssems[p]).wait()
        if with_deg:
            pltpu.make_async_copy(ones_v, accd.at[ivs[p]], ssems[p]).wait()

    load(0, 0)
    load(1, 1)

    @pl.loop(0, nw, step=2)
    def _(g):
        for p in range(2):
            w = g + p
            wait_load(p)

            @pl.loop(0, K, step=16)
            def _(q):
                dl = dvs[p][pl.ds(q, 16)] - half_base
                valid = (dl >= 0) & (dl < H)
                ivs[p][pl.ds(q, 16)] = jnp.where(valid, dl, H + (dl & (NDUMP - 1)))

            @pl.when(w >= 2)
            def _():
                wait_scatter(p)

            pltpu.async_copy(h_hbm.at[svs[p]], rows_[p], gsem).wait()
            pltpu.async_copy(rows_[p], acc.at[ivs[p]], ssems[p], add=True)
            if with_deg:
                pltpu.async_copy(ones_v, accd.at[ivs[p]], ssems[p], add=True)

            @pl.when(w + 2 < nw)
            def _():
                load(w + 2, p)

    wait_scatter(0)
    wait_scatter(1)
    plsc.subcore_barrier()
    out_base = core * H + sub * (H // NSUB)
    pltpu.sync_copy(acc.at[pl.ds(sub * (H // NSUB), H // NSUB)],
                    agg_hbm.at[pl.ds(out_base, H // NSUB)])
    if with_deg:
        pltpu.sync_copy(accd.at[pl.ds(sub * (H // NSUB), H // NSUB)],
                        deg_hbm.at[pl.ds(out_base, H // NSUB)])


@jax.jit
def _aggregate_deg(h, srcp, dstp, z2, z1):
    kfn = pl.kernel(
        functools.partial(_agg_body, True),
        out_type=(jax.ShapeDtypeStruct((N_pad, F), jnp.float32),
                  jax.ShapeDtypeStruct((N_pad,), jnp.float32)),
        mesh=_mesh,
        compiler_params=_sc_params,
        scratch_types=[
            pltpu.VMEM((K,), jnp.int32),
            pltpu.VMEM((K,), jnp.int32),
            pltpu.VMEM((K,), jnp.int32),
            pltpu.VMEM((K, F), jnp.float32),
            pltpu.VMEM((K,), jnp.int32),
            pltpu.VMEM((K,), jnp.int32),
            pltpu.VMEM((K,), jnp.int32),
            pltpu.VMEM((K, F), jnp.float32),
            pltpu.SemaphoreType.DMA,
            pltpu.SemaphoreType.DMA,
            pltpu.SemaphoreType.DMA,
            pltpu.SemaphoreType.DMA,
            pltpu.SemaphoreType.DMA,
            pltpu.VMEM((K,), jnp.float32),
            pltpu.VMEM_SHARED((H2, F), jnp.float32),
            pltpu.VMEM_SHARED((H2,), jnp.float32),
        ],
    )
    return kfn(h, srcp, dstp, z2, z1)


@jax.jit
def _aggregate(h, srcp, dstp, z2):
    kfn = pl.kernel(
        functools.partial(_agg_body, False),
        out_type=jax.ShapeDtypeStruct((N_pad, F), jnp.float32),
        mesh=_mesh,
        compiler_params=_sc_params,
        scratch_types=[
            pltpu.VMEM((K,), jnp.int32),
            pltpu.VMEM((K,), jnp.int32),
            pltpu.VMEM((K,), jnp.int32),
            pltpu.VMEM((K, F), jnp.float32),
            pltpu.VMEM((K,), jnp.int32),
            pltpu.VMEM((K,), jnp.int32),
            pltpu.VMEM((K,), jnp.int32),
            pltpu.VMEM((K, F), jnp.float32),
            pltpu.SemaphoreType.DMA,
            pltpu.SemaphoreType.DMA,
            pltpu.SemaphoreType.DMA,
            pltpu.SemaphoreType.DMA,
            pltpu.SemaphoreType.DMA,
            pltpu.VMEM_SHARED((H2, F), jnp.float32),
        ],
    )
    return kfn(h, srcp, dstp, z2)


def _dense_body(agg_ref, deg_ref, h_ref, wl_ref, b_ref, wr_ref, out_ref):
    mean = agg_ref[...] / jnp.maximum(deg_ref[...], 1.0)[:, None]
    out = (lax.dot_general(mean, wl_ref[...], (((1,), (1,)), ((), ())),
                           preferred_element_type=jnp.float32)
           + lax.dot_general(h_ref[...], wr_ref[...], (((1,), (1,)), ((), ())),
                             preferred_element_type=jnp.float32)
           + b_ref[...])
    out_ref[...] = jnp.maximum(out, 0.0)


@jax.jit
def _dense(agg, deg, h, wl, b, wr):
    return pl.pallas_call(
        _dense_body,
        grid=(NBLK,),
        in_specs=[
            pl.BlockSpec((BN, F), lambda i: (i, 0)),
            pl.BlockSpec((BN,), lambda i: (i,)),
            pl.BlockSpec((BN, F), lambda i: (i, 0)),
            pl.BlockSpec((F, F), lambda i: (0, 0)),
            pl.BlockSpec((1, F), lambda i: (0, 0)),
            pl.BlockSpec((F, F), lambda i: (0, 0)),
        ],
        out_specs=pl.BlockSpec((BN, F), lambda i: (i, 0)),
        out_shape=jax.ShapeDtypeStruct((N_pad, F), jnp.float32),
    )(agg, deg, h, wl, b, wr)


def _pool_body(h_ref, batch_ref, wc_ref, bc_ref, out_ref, acc_s, acc_c):
    i = pl.program_id(0)

    @pl.when(i == 0)
    def _():
        acc_s[...] = jnp.zeros_like(acc_s)
        acc_c[...] = jnp.zeros_like(acc_c)

    ids = batch_ref[...]
    gi = lax.broadcasted_iota(jnp.int32, (G, BN), 0)
    oh = (gi == ids[None, :]).astype(jnp.float32)
    acc_s[...] += lax.dot_general(oh, h_ref[...], (((1,), (0,)), ((), ())),
                                  preferred_element_type=jnp.float32)
    acc_c[...] += jnp.sum(oh, axis=1)

    @pl.when(i == NBLK - 1)
    def _():
        hg = acc_s[...] / jnp.maximum(acc_c[...], 1.0)[:, None]
        out_ref[...] = lax.dot_general(hg, wc_ref[...], (((1,), (1,)), ((), ())),
                                       preferred_element_type=jnp.float32) + bc_ref[...]


@jax.jit
def _pool(h, batch, wc, bc):
    return pl.pallas_call(
        _pool_body,
        grid=(NBLK,),
        in_specs=[
            pl.BlockSpec((BN, F), lambda i: (i, 0)),
            pl.BlockSpec((BN,), lambda i: (i,)),
            pl.BlockSpec((NCLS, F), lambda i: (0, 0)),
            pl.BlockSpec((1, NCLS), lambda i: (0, 0)),
        ],
        out_specs=pl.BlockSpec((G, NCLS), lambda i: (0, 0)),
        out_shape=jax.ShapeDtypeStruct((G, NCLS), jnp.float32),
        scratch_shapes=[
            pltpu.VMEM((G, F), jnp.float32),
            pltpu.VMEM((G,), jnp.float32),
        ],
    )(h, batch, wc, bc)


def kernel(x, edge_index, batch, shape_emb, color_emb, W1l, b1, W1r, W2l, b2, W2r, Wc, bc):
    x = x.astype(jnp.int32)
    x0 = jnp.pad(x[:, 0], (0, N_pad - N))
    x1 = jnp.pad(x[:, 1], (0, N_pad - N))
    src = jnp.pad(edge_index[0], (0, E_pad - E))
    dst = jnp.pad(edge_index[1], (0, E_pad - E), constant_values=-1)
    batch_p = jnp.pad(batch, (0, N_pad - N), constant_values=G)
    z2 = jnp.zeros((ZCH, F), jnp.float32)
    z1 = jnp.zeros((ZCH,), jnp.float32)

    h0 = _embed(x0, x1, shape_emb, color_emb)
    a1, deg = _aggregate_deg(h0, src, dst, z2, z1)
    h1 = _dense(a1, deg, h0, W1l, b1.reshape(1, F), W1r)
    a2 = _aggregate(h1, src, dst, z2)
    h2 = _dense(a2, deg, h1, W2l, b2.reshape(1, F), W2r)
    return _pool(h2, batch_p, Wc, bc.reshape(1, NCLS))
